# Initial kernel scaffold; baseline (speedup 1.0000x reference)
import functools

import jax
import jax.numpy as jnp
from jax import lax
from jax.experimental import pallas as pl
from jax.experimental.pallas import tpu as pltpu
from jax.experimental.pallas import tpu_sc as plsc

N = 10000
E = 320000
CH_IN = 128
CH_OUT = 128

NC = 2
NS = 16
NW = NC * NS
PT = E // NW
CH = 80
NCHUNK = PT // CH
RPT = N // NS

_MESH = plsc.VectorSubcoreMesh(core_axis_name="c", subcore_axis_name="s")


@functools.partial(
    pl.kernel,
    out_type=jax.ShapeDtypeStruct((NC, N, 16), jnp.float32),
    mesh=_MESH,
    scratch_types=[
        pltpu.VMEM((CH,), jnp.int32),
        pltpu.VMEM((CH, 16), jnp.float32),
        pltpu.VMEM_SHARED((N, 16), jnp.float32),
    ],
)
def _sc_degree(dst_hbm, ones_hbm, z16_hbm, out_hbm, idx_v, ones_v, deg_sh):
    cid = lax.axis_index("c")
    sid = lax.axis_index("s")
    wid = sid * NC + cid
    r0 = sid * RPT

    pltpu.sync_copy(ones_hbm, ones_v)
    pltpu.sync_copy(z16_hbm, deg_sh.at[pl.ds(r0, RPT)])
    plsc.subcore_barrier()

    base = wid * PT

    @pl.loop(0, NCHUNK)
    def _(c):
        pltpu.sync_copy(dst_hbm.at[pl.ds(base + c * CH, CH)], idx_v)
        pltpu.sync_copy(ones_v, deg_sh.at[idx_v], add=True)

    plsc.subcore_barrier()
    pltpu.sync_copy(deg_sh.at[pl.ds(r0, RPT)], out_hbm.at[cid].at[pl.ds(r0, RPT)])


@functools.partial(
    pl.kernel,
    out_type=jax.ShapeDtypeStruct((NC, N, CH_OUT), jnp.float32),
    mesh=_MESH,
    scratch_types=[
        pltpu.VMEM((CH,), jnp.int32),
        pltpu.VMEM((CH,), jnp.int32),
        pltpu.VMEM((CH, CH_OUT), jnp.float32),
        pltpu.VMEM_SHARED((N, CH_OUT), jnp.float32),
        pltpu.SemaphoreType.DMA,
    ],
)
def _sc_aggregate(y_hbm, src_hbm, dst_hbm, z128_hbm, out_hbm,
                  src_v, dst_v, rows_v, acc_sh, sem):
    cid = lax.axis_index("c")
    sid = lax.axis_index("s")
    wid = sid * NC + cid
    r0 = sid * RPT

    pltpu.sync_copy(z128_hbm, acc_sh.at[pl.ds(r0, RPT)])
    plsc.subcore_barrier()

    base = wid * PT

    @pl.loop(0, NCHUNK)
    def _(c):
        off = base + c * CH
        pltpu.sync_copy(src_hbm.at[pl.ds(off, CH)], src_v)
        pltpu.sync_copy(dst_hbm.at[pl.ds(off, CH)], dst_v)
        pltpu.async_copy(y_hbm.at[src_v], rows_v, sem).wait()
        pltpu.sync_copy(rows_v, acc_sh.at[dst_v], add=True)

    plsc.subcore_barrier()
    pltpu.sync_copy(acc_sh.at[pl.ds(r0, RPT)], out_hbm.at[cid].at[pl.ds(r0, RPT)])


def _mm_body(x_ref, w_ref, o_ref):
    o_ref[...] = jnp.dot(x_ref[...], w_ref[...],
                         preferred_element_type=jnp.float32)


def _prep_body(xw_ref, degp_ref, y_ref):
    deg = 1.0 + degp_ref[0, :, 0] + degp_ref[1, :, 0]
    dis = lax.rsqrt(deg)
    y_ref[...] = xw_ref[...] * dis[:, None]


def _out_body(accp_ref, y_ref, degp_ref, b_ref, o_ref):
    deg = 1.0 + degp_ref[0, :, 0] + degp_ref[1, :, 0]
    dis = lax.rsqrt(deg)
    s = accp_ref[0] + accp_ref[1] + y_ref[...]
    o_ref[...] = s * dis[:, None] + b_ref[...]


def kernel(x, edge_index, W, b):
    ei = edge_index.astype(jnp.int32)
    src = ei[0]
    dst = ei[1]
    ones16 = jnp.ones((CH, 16), jnp.float32)
    z16 = jnp.zeros((RPT, 16), jnp.float32)
    z128 = jnp.zeros((RPT, CH_OUT), jnp.float32)
    b2 = b.reshape(1, CH_OUT).astype(jnp.float32)

    xw = pl.pallas_call(
        _mm_body,
        out_shape=jax.ShapeDtypeStruct((N, CH_OUT), jnp.float32),
        grid=(8,),
        in_specs=[
            pl.BlockSpec((N // 8, CH_IN), lambda i: (i, 0)),
            pl.BlockSpec((CH_IN, CH_OUT), lambda i: (0, 0)),
        ],
        out_specs=pl.BlockSpec((N // 8, CH_OUT), lambda i: (i, 0)),
    )(x, W)

    degp = _sc_degree(dst, ones16, z16)

    y = pl.pallas_call(
        _prep_body,
        out_shape=jax.ShapeDtypeStruct((N, CH_OUT), jnp.float32),
    )(xw, degp)

    accp = _sc_aggregate(y, src, dst, z128)

    out = pl.pallas_call(
        _out_body,
        out_shape=jax.ShapeDtypeStruct((N, CH_OUT), jnp.float32),
    )(accp, y, degp, b2)
    return out


# speedup vs baseline: 16.1446x; 16.1446x over previous
import functools

import jax
import jax.numpy as jnp
from jax import lax
from jax.experimental import pallas as pl
from jax.experimental.pallas import tpu as pltpu
from jax.experimental.pallas import tpu_sc as plsc

N = 10000
E = 320000
CH_IN = 128
CH_OUT = 128

NC = 2
NS = 16
NW = NC * NS
PT = E // NW
CH = 80
NCHUNK = PT // CH
RPT = 632
R_LAST = N - RPT

_MESH = plsc.VectorSubcoreMesh(core_axis_name="c", subcore_axis_name="s")


@functools.partial(
    pl.kernel,
    out_type=jax.ShapeDtypeStruct((NC, N, 128), jnp.float32),
    mesh=_MESH,
    scratch_types=[
        pltpu.VMEM((CH,), jnp.int32),
        pltpu.VMEM((CH, 128), jnp.float32),
        pltpu.VMEM_SHARED((N, 128), jnp.float32),
    ],
)
def _sc_degree(dst_hbm, ones_hbm, z16_hbm, out_hbm, idx_v, ones_v, deg_sh):
    cid = lax.axis_index("c")
    sid = lax.axis_index("s")
    wid = sid * NC + cid
    r0 = jnp.minimum(sid * RPT, R_LAST)

    pltpu.sync_copy(ones_hbm, ones_v)
    pltpu.sync_copy(z16_hbm, deg_sh.at[pl.ds(r0, RPT)])
    plsc.subcore_barrier()

    base = wid * PT

    @pl.loop(0, NCHUNK)
    def _(c):
        pltpu.sync_copy(dst_hbm.at[pl.ds(base + c * CH, CH)], idx_v)
        pltpu.sync_copy(ones_v, deg_sh.at[idx_v], add=True)

    plsc.subcore_barrier()
    pltpu.sync_copy(deg_sh.at[pl.ds(r0, RPT)], out_hbm.at[cid].at[pl.ds(r0, RPT)])


@functools.partial(
    pl.kernel,
    out_type=jax.ShapeDtypeStruct((NC, N, CH_OUT), jnp.float32),
    mesh=_MESH,
    scratch_types=[
        pltpu.VMEM((CH,), jnp.int32),
        pltpu.VMEM((CH,), jnp.int32),
        pltpu.VMEM((CH, CH_OUT), jnp.float32),
        pltpu.VMEM_SHARED((N, CH_OUT), jnp.float32),
        pltpu.SemaphoreType.DMA,
    ],
)
def _sc_aggregate(y_hbm, src_hbm, dst_hbm, z128_hbm, out_hbm,
                  src_v, dst_v, rows_v, acc_sh, sem):
    cid = lax.axis_index("c")
    sid = lax.axis_index("s")
    wid = sid * NC + cid
    r0 = jnp.minimum(sid * RPT, R_LAST)

    pltpu.sync_copy(z128_hbm, acc_sh.at[pl.ds(r0, RPT)])
    plsc.subcore_barrier()

    base = wid * PT

    @pl.loop(0, NCHUNK)
    def _(c):
        off = base + c * CH
        pltpu.sync_copy(src_hbm.at[pl.ds(off, CH)], src_v)
        pltpu.sync_copy(dst_hbm.at[pl.ds(off, CH)], dst_v)
        pltpu.async_copy(y_hbm.at[src_v], rows_v, sem).wait()
        pltpu.sync_copy(rows_v, acc_sh.at[dst_v], add=True)

    plsc.subcore_barrier()
    pltpu.sync_copy(acc_sh.at[pl.ds(r0, RPT)], out_hbm.at[cid].at[pl.ds(r0, RPT)])


def _mm_body(x_ref, w_ref, o_ref):
    o_ref[...] = jnp.dot(x_ref[...], w_ref[...],
                         preferred_element_type=jnp.float32)


def _prep_body(xw_ref, degp_ref, y_ref):
    deg = 1.0 + degp_ref[0, :, 0] + degp_ref[1, :, 0]
    dis = lax.rsqrt(deg)
    y_ref[...] = xw_ref[...] * dis[:, None]


def _out_body(accp_ref, y_ref, degp_ref, b_ref, o_ref):
    deg = 1.0 + degp_ref[0, :, 0] + degp_ref[1, :, 0]
    dis = lax.rsqrt(deg)
    s = accp_ref[0] + accp_ref[1] + y_ref[...]
    o_ref[...] = s * dis[:, None] + b_ref[...]


def kernel(x, edge_index, W, b):
    ei = edge_index.astype(jnp.int32)
    src = ei[0]
    dst = ei[1]
    ones128 = jnp.ones((CH, 128), jnp.float32)
    z128 = jnp.zeros((RPT, CH_OUT), jnp.float32)
    b2 = b.reshape(1, CH_OUT).astype(jnp.float32)

    xw = pl.pallas_call(
        _mm_body,
        out_shape=jax.ShapeDtypeStruct((N, CH_OUT), jnp.float32),
        grid=(10,),
        in_specs=[
            pl.BlockSpec((N // 10, CH_IN), lambda i: (i, 0)),
            pl.BlockSpec((CH_IN, CH_OUT), lambda i: (0, 0)),
        ],
        out_specs=pl.BlockSpec((N // 10, CH_OUT), lambda i: (i, 0)),
    )(x, W)

    degp = _sc_degree(dst, ones128, z128)

    y = pl.pallas_call(
        _prep_body,
        out_shape=jax.ShapeDtypeStruct((N, CH_OUT), jnp.float32),
    )(xw, degp)

    accp = _sc_aggregate(y, src, dst, z128)

    out = pl.pallas_call(
        _out_body,
        out_shape=jax.ShapeDtypeStruct((N, CH_OUT), jnp.float32),
    )(accp, y, degp, b2)
    return out


# speedup vs baseline: 38.6751x; 2.3955x over previous
import dataclasses
import functools

import jax
import jax.numpy as jnp
from jax import lax
from jax.experimental import pallas as pl
from jax.experimental.pallas import tpu as pltpu
from jax.experimental.pallas import tpu_sc as plsc

N = 10000
E = 320000
CH_IN = 128
CH_OUT = 128

NC = 2
NS = 16
NW = NC * NS
PT = E // NW
CH = 80
NCHUNK = PT // CH
RING = 3
RPT = 632
R_LAST = N - RPT

HB = 10240
HROWS = HB // 128

_MESH = plsc.VectorSubcoreMesh(core_axis_name="c", subcore_axis_name="s")

_CP = pltpu.CompilerParams()
if "needs_layout_passes" in pltpu.CompilerParams.__dataclass_fields__:
    _CP = dataclasses.replace(_CP, needs_layout_passes=False)


@functools.partial(
    pl.kernel,
    out_type=jax.ShapeDtypeStruct((NC, HROWS, 128), jnp.float32),
    mesh=_MESH,
    scratch_types=[
        pltpu.VMEM((PT,), jnp.int32),
        pltpu.VMEM((HROWS, 128), jnp.float32),
        pltpu.VMEM((HROWS,), jnp.int32),
        pltpu.VMEM_SHARED((HROWS, 128), jnp.float32),
    ],
    compiler_params=_CP,
)
def _sc_degree(dst_hbm, zrow_hbm, out_hbm, dstb, hist, iota_v, deg_sh):
    cid = lax.axis_index("c")
    sid = lax.axis_index("s")
    wid = sid * NC + cid

    @pl.when(sid == 0)
    def _():
        pltpu.sync_copy(zrow_hbm, deg_sh)

    zero16 = jnp.zeros((16,), jnp.float32)

    @pl.loop(0, HROWS)
    def _(r):
        @pl.loop(0, 8)
        def _(k):
            hist[r, pl.ds(k * 16, 16)] = zero16

    @pl.loop(0, HROWS // 16)
    def _(k):
        iota_v[pl.ds(k * 16, 16)] = lax.iota(jnp.int32, 16) + k * 16

    pltpu.sync_copy(dst_hbm.at[wid], dstb)

    @pl.loop(0, PT // 16)
    def _(i):
        v = dstb[pl.ds(i * 16, 16)]
        cnt, last = plsc.scan_count(v)
        row = lax.shift_right_logical(v, 7)
        col = lax.bitwise_and(v, 127)
        plsc.addupdate_scatter(hist, [row, col], cnt.astype(jnp.float32),
                               mask=last)

    plsc.subcore_barrier()
    pltpu.sync_copy(hist, deg_sh.at[iota_v], add=True)
    plsc.subcore_barrier()

    @pl.when(sid == 0)
    def _():
        pltpu.sync_copy(deg_sh, out_hbm.at[cid])


@functools.partial(
    pl.kernel,
    out_type=jax.ShapeDtypeStruct((NC, N, CH_OUT), jnp.float32),
    mesh=_MESH,
    scratch_types=[
        pltpu.VMEM((RING, CH), jnp.int32),
        pltpu.VMEM((RING, CH), jnp.int32),
        [pltpu.VMEM((CH, CH_OUT), jnp.float32)] * RING,
        pltpu.VMEM_SHARED((N, CH_OUT), jnp.float32),
        [pltpu.SemaphoreType.DMA] * RING,
        [pltpu.SemaphoreType.DMA] * RING,
        [pltpu.SemaphoreType.DMA] * RING,
    ],
)
def _sc_aggregate(y_hbm, src_hbm, dst_hbm, z128_hbm, out_hbm,
                  srcv, dstv, rows, acc_sh, gsem, ssem, isem):
    cid = lax.axis_index("c")
    sid = lax.axis_index("s")
    wid = sid * NC + cid
    r0 = jnp.minimum(sid * RPT, R_LAST)

    src_t = src_hbm.at[wid]
    dst_t = dst_hbm.at[wid]

    pltpu.sync_copy(z128_hbm, acc_sh.at[pl.ds(r0, RPT)])
    plsc.subcore_barrier()

    def istart(c, b):
        pltpu.async_copy(src_t.at[c], srcv.at[b], isem[b])
        pltpu.async_copy(dst_t.at[c], dstv.at[b], isem[b])

    def iwait(b):
        pltpu.make_async_copy(src_t.at[0], srcv.at[b], isem[b]).wait()
        pltpu.make_async_copy(dst_t.at[0], dstv.at[b], isem[b]).wait()

    def gstart(b):
        pltpu.async_copy(y_hbm.at[srcv.at[b]], rows[b], gsem[b])

    def gwait(b):
        pltpu.make_async_copy(y_hbm.at[srcv.at[b]], rows[b], gsem[b]).wait()

    def sstart(b):
        pltpu.async_copy(rows[b], acc_sh.at[dstv.at[b]], ssem[b], add=True)

    def swait(b):
        pltpu.make_async_copy(rows[b], acc_sh.at[dstv.at[b]], ssem[b]).wait()

    for b in range(RING):
        istart(b, b)

    NMAIN = NCHUNK // RING * RING

    @pl.loop(0, NMAIN // RING)
    def _(q):
        c0 = q * RING
        for b in range(RING):
            iwait(b)
            gstart(b)
        for b in range(RING):
            gwait(b)
            sstart(b)
        for b in range(RING):
            swait(b)

            @pl.when(c0 + RING + b < NCHUNK)
            def _():
                istart(c0 + RING + b, b)

    for b in range(NCHUNK - NMAIN):
        iwait(b)
        gstart(b)
    for b in range(NCHUNK - NMAIN):
        gwait(b)
        sstart(b)
    for b in range(NCHUNK - NMAIN):
        swait(b)

    plsc.subcore_barrier()
    pltpu.sync_copy(acc_sh.at[pl.ds(r0, RPT)], out_hbm.at[cid].at[pl.ds(r0, RPT)])


def _mm_body(x_ref, w_ref, o_ref):
    o_ref[...] = jnp.dot(x_ref[...], w_ref[...],
                         preferred_element_type=jnp.float32)


def _prep_body(xw_ref, degp_ref, y_ref):
    deg = 1.0 + (degp_ref[0] + degp_ref[1]).reshape(HB)[:N]
    dis = lax.rsqrt(deg)
    y_ref[...] = xw_ref[...] * dis[:, None]


def _out_body(accp_ref, y_ref, degp_ref, b_ref, o_ref):
    deg = 1.0 + (degp_ref[0] + degp_ref[1]).reshape(HB)[:N]
    dis = lax.rsqrt(deg)
    s = accp_ref[0] + accp_ref[1] + y_ref[...]
    o_ref[...] = s * dis[:, None] + b_ref[...]


def kernel(x, edge_index, W, b):
    ei = edge_index.astype(jnp.int32)
    src3 = ei[0].reshape(NW, NCHUNK, CH)
    dst3 = ei[1].reshape(NW, NCHUNK, CH)
    dst2 = ei[1].reshape(NW, PT)
    zrow = jnp.zeros((HROWS, 128), jnp.float32)
    z128 = jnp.zeros((RPT, CH_OUT), jnp.float32)
    b2 = b.reshape(1, CH_OUT).astype(jnp.float32)

    xw = pl.pallas_call(
        _mm_body,
        out_shape=jax.ShapeDtypeStruct((N, CH_OUT), jnp.float32),
        grid=(10,),
        in_specs=[
            pl.BlockSpec((N // 10, CH_IN), lambda i: (i, 0)),
            pl.BlockSpec((CH_IN, CH_OUT), lambda i: (0, 0)),
        ],
        out_specs=pl.BlockSpec((N // 10, CH_OUT), lambda i: (i, 0)),
    )(x, W)

    degp = _sc_degree(dst2, zrow)

    y = pl.pallas_call(
        _prep_body,
        out_shape=jax.ShapeDtypeStruct((N, CH_OUT), jnp.float32),
    )(xw, degp)

    accp = _sc_aggregate(y, src3, dst3, z128)

    out = pl.pallas_call(
        _out_body,
        out_shape=jax.ShapeDtypeStruct((N, CH_OUT), jnp.float32),
    )(accp, y, degp, b2)
    return out


# speedup vs baseline: 40.0237x; 1.0349x over previous
import dataclasses
import functools

import jax
import jax.numpy as jnp
from jax import lax
from jax.experimental import pallas as pl
from jax.experimental.pallas import tpu as pltpu
from jax.experimental.pallas import tpu_sc as plsc

N = 10000
E = 320000
CH_IN = 128
CH_OUT = 128

NC = 2
NS = 16
NW = NC * NS
PT = E // NW
CH = 80
NCHUNK = PT // CH
RING = 3
RPT = 632
R_LAST = N - RPT

HB = 10240
HROWS = HB // 128

_MESH = plsc.VectorSubcoreMesh(core_axis_name="c", subcore_axis_name="s")

_CP = pltpu.CompilerParams()
if "needs_layout_passes" in pltpu.CompilerParams.__dataclass_fields__:
    _CP = dataclasses.replace(_CP, needs_layout_passes=False)


@functools.partial(
    pl.kernel,
    out_type=jax.ShapeDtypeStruct((NC, HROWS, 128), jnp.float32),
    mesh=_MESH,
    scratch_types=[
        pltpu.VMEM((PT,), jnp.int32),
        pltpu.VMEM((HROWS, 128), jnp.float32),
        pltpu.VMEM((HROWS,), jnp.int32),
        pltpu.VMEM_SHARED((HROWS, 128), jnp.float32),
    ],
    compiler_params=_CP,
)
def _sc_degree(dst_hbm, zrow_hbm, out_hbm, dstb, hist, iota_v, deg_sh):
    cid = lax.axis_index("c")
    sid = lax.axis_index("s")
    wid = sid * NC + cid

    @pl.when(sid == 0)
    def _():
        pltpu.sync_copy(zrow_hbm, deg_sh)

    zero16 = jnp.zeros((16,), jnp.float32)

    @pl.loop(0, HROWS)
    def _(r):
        @pl.loop(0, 8)
        def _(k):
            hist[r, pl.ds(k * 16, 16)] = zero16

    @pl.loop(0, HROWS // 16)
    def _(k):
        iota_v[pl.ds(k * 16, 16)] = lax.iota(jnp.int32, 16) + k * 16

    pltpu.sync_copy(dst_hbm.at[wid], dstb)

    @pl.loop(0, PT // 16)
    def _(i):
        v = dstb[pl.ds(i * 16, 16)]
        cnt, last = plsc.scan_count(v)
        row = lax.shift_right_logical(v, 7)
        col = lax.bitwise_and(v, 127)
        plsc.addupdate_scatter(hist, [row, col], cnt.astype(jnp.float32),
                               mask=last)

    plsc.subcore_barrier()
    pltpu.sync_copy(hist, deg_sh.at[iota_v], add=True)
    plsc.subcore_barrier()

    @pl.when(sid == 0)
    def _():
        pltpu.sync_copy(deg_sh, out_hbm.at[cid])


@functools.partial(
    pl.kernel,
    out_type=jax.ShapeDtypeStruct((NC, N, CH_OUT), jnp.float32),
    mesh=_MESH,
    scratch_types=[
        pltpu.VMEM((2 * RING, CH), jnp.int32),
        pltpu.VMEM((2 * RING, CH), jnp.int32),
        [pltpu.VMEM((CH, CH_OUT), jnp.float32)] * RING,
        pltpu.VMEM_SHARED((N, CH_OUT), jnp.float32),
        [pltpu.SemaphoreType.DMA] * RING,
        [pltpu.SemaphoreType.DMA] * RING,
        [pltpu.SemaphoreType.DMA] * (2 * RING),
    ],
)
def _sc_aggregate(y_hbm, src_hbm, dst_hbm, z128_hbm, out_hbm,
                  srcv, dstv, rows, acc_sh, gsem, ssem, isem):
    cid = lax.axis_index("c")
    sid = lax.axis_index("s")
    wid = sid * NC + cid
    r0 = jnp.minimum(sid * RPT, R_LAST)

    src_t = src_hbm.at[wid]
    dst_t = dst_hbm.at[wid]

    pltpu.sync_copy(z128_hbm, acc_sh.at[pl.ds(r0, RPT)])
    plsc.subcore_barrier()

    def istart(c, i):
        pltpu.async_copy(src_t.at[c], srcv.at[i], isem[i])
        pltpu.async_copy(dst_t.at[c], dstv.at[i], isem[i])

    def iwait(i):
        pltpu.make_async_copy(src_t.at[0], srcv.at[i], isem[i]).wait()
        pltpu.make_async_copy(dst_t.at[0], dstv.at[i], isem[i]).wait()

    def gstart(b, i):
        pltpu.async_copy(y_hbm.at[srcv.at[i]], rows[b], gsem[b])

    def gwait(b):
        pltpu.make_async_copy(y_hbm.at[srcv.at[0]], rows[b], gsem[b]).wait()

    def sstart(b, i):
        pltpu.async_copy(rows[b], acc_sh.at[dstv.at[i]], ssem[b], add=True)

    def swait(b):
        pltpu.make_async_copy(rows[b], acc_sh.at[dstv.at[0]], ssem[b]).wait()

    GRP = 2 * RING
    for i in range(GRP):
        istart(i, i)

    NMAIN = NCHUNK // GRP * GRP

    @pl.loop(0, NMAIN // GRP)
    def _(q):
        c0 = q * GRP
        for b in range(RING):
            iwait(b)
            gstart(b, b)
        for b in range(RING):
            gwait(b)
            sstart(b, b)
        for b in range(RING):
            swait(b)
            istart(c0 + GRP + b, b)
            iwait(b + RING)
            gstart(b, b + RING)
        for b in range(RING):
            gwait(b)
            sstart(b, b + RING)
        for b in range(RING):
            swait(b)

            @pl.when(c0 + GRP + RING + b < NCHUNK)
            def _():
                istart(c0 + GRP + RING + b, b + RING)

    rem = NCHUNK - NMAIN
    w1 = min(rem, RING)
    for j in range(w1):
        iwait(j)
        gstart(j, j)
    for j in range(w1):
        gwait(j)
        sstart(j, j)
    for j in range(RING, rem):
        b = j - RING
        swait(b)
        iwait(j)
        gstart(b, j)
    for j in range(RING, rem):
        b = j - RING
        gwait(b)
        sstart(b, j)
    for b in range(w1):
        swait(b)

    plsc.subcore_barrier()
    pltpu.sync_copy(acc_sh.at[pl.ds(r0, RPT)], out_hbm.at[cid].at[pl.ds(r0, RPT)])


def _prep_body(x_ref, w_ref, degp_ref, y_ref):
    deg = 1.0 + (degp_ref[0] + degp_ref[1]).reshape(HB)[:N]
    dis = lax.rsqrt(deg)
    xw = jnp.dot(x_ref[...], w_ref[...], preferred_element_type=jnp.float32)
    y_ref[...] = xw * dis[:, None]


def _out_body(accp_ref, y_ref, degp_ref, b_ref, o_ref):
    deg = 1.0 + (degp_ref[0] + degp_ref[1]).reshape(HB)[:N]
    dis = lax.rsqrt(deg)
    s = accp_ref[0] + accp_ref[1] + y_ref[...]
    o_ref[...] = s * dis[:, None] + b_ref[...]


def kernel(x, edge_index, W, b):
    ei = edge_index.astype(jnp.int32)
    src3 = ei[0].reshape(NW, NCHUNK, CH)
    dst3 = ei[1].reshape(NW, NCHUNK, CH)
    dst2 = ei[1].reshape(NW, PT)
    zrow = jnp.zeros((HROWS, 128), jnp.float32)
    z128 = jnp.zeros((RPT, CH_OUT), jnp.float32)
    b2 = b.reshape(1, CH_OUT).astype(jnp.float32)

    degp = _sc_degree(dst2, zrow)

    y = pl.pallas_call(
        _prep_body,
        out_shape=jax.ShapeDtypeStruct((N, CH_OUT), jnp.float32),
    )(x, W, degp)

    accp = _sc_aggregate(y, src3, dst3, z128)

    out = pl.pallas_call(
        _out_body,
        out_shape=jax.ShapeDtypeStruct((N, CH_OUT), jnp.float32),
    )(accp, y, degp, b2)
    return out
